# baseline (speedup 1.0000x reference)
import functools

import jax
import jax.numpy as jnp
import numpy as np
from jax import lax
from jax.experimental import pallas as pl
from jax.experimental.pallas import tpu as pltpu
from jax.experimental.pallas import tpu_sc as plsc

N = 10000
E = 320000
D = 128
L = 16
NC = 2
NS = 16
NW = NC * NS
E_PER_W = E // NW
C = 80
CHUNKS = E_PER_W // C
N_PER_TILE = N // NS
DEN_PAD = 10240
ZDEN = DEN_PAD // NS

_PERM = np.empty((D,), dtype=np.int32)
for _q in range(D // 32):
    for _k in range(16):
        _PERM[32 * _q + 2 * _k] = 32 * _q + _k
        _PERM[32 * _q + 2 * _k + 1] = 32 * _q + 16 + _k


def _project(x, W_src, W_dst):
    BLK = 1000

    def body(x_ref, ws_ref, wd_ref, xs_ref, xd_ref):
        xb = x_ref[...]
        xs_ref[...] = jnp.dot(xb, ws_ref[...], preferred_element_type=jnp.float32)
        xd_ref[...] = jnp.dot(xb, wd_ref[...], preferred_element_type=jnp.float32)

    return pl.pallas_call(
        body,
        grid=(N // BLK,),
        in_specs=[
            pl.BlockSpec((BLK, D), lambda i: (i, 0)),
            pl.BlockSpec((D, D), lambda i: (0, 0)),
            pl.BlockSpec((D, D), lambda i: (0, 0)),
        ],
        out_specs=[
            pl.BlockSpec((BLK, D), lambda i: (i, 0)),
            pl.BlockSpec((BLK, D), lambda i: (i, 0)),
        ],
        out_shape=[jax.ShapeDtypeStruct((N, D), jnp.float32)] * 2,
    )(x, W_src, W_dst)


def _sc_edge_pass(xs, xd, src3, dst3, att):
    mesh = plsc.VectorSubcoreMesh(core_axis_name="c", subcore_axis_name="s")

    @functools.partial(
        pl.kernel,
        out_type=[
            jax.ShapeDtypeStruct((NC, N, D), jnp.float32),
            jax.ShapeDtypeStruct((NC, DEN_PAD), jnp.float32),
        ],
        mesh=mesh,
        compiler_params=pltpu.CompilerParams(
            needs_layout_passes=False, use_tc_tiling_on_sc=False),
        scratch_types=[
            pltpu.VMEM((C,), jnp.int32),
            pltpu.VMEM((C,), jnp.int32),
            pltpu.VMEM((C,), jnp.int32),
            pltpu.VMEM((C,), jnp.int32),
            pltpu.VMEM((C, D), jnp.float32),
            pltpu.VMEM((C, D), jnp.float32),
            pltpu.VMEM((C, D // 2), jnp.int32),
            pltpu.VMEM((C, D // 2), jnp.int32),
            pltpu.VMEM((C,), jnp.float32),
            pltpu.VMEM((C,), jnp.float32),
            pltpu.VMEM((D,), jnp.float32),
            pltpu.VMEM_SHARED((N, D), jnp.float32),
            pltpu.VMEM_SHARED((DEN_PAD,), jnp.float32),
            pltpu.SemaphoreType.DMA,
            pltpu.SemaphoreType.DMA,
            pltpu.SemaphoreType.DMA,
            pltpu.SemaphoreType.DMA,
        ],
    )
    def edge_kernel(xs_hbm, xd_hbm, src_hbm, dst_hbm, att_hbm,
                    acc_out, den_out,
                    srcc0, srcc1, dstc0, dstc1, xsr0, xsr1, xdr0, xdr1,
                    exv0, exv1, att_v, acc_sh, den_sh,
                    semg0, semg1, semi0, semi1):
        cid = lax.axis_index("c")
        sid = lax.axis_index("s")
        wid = sid * NC + cid

        bufs = ((srcc0, dstc0, xsr0, xdr0, exv0, semg0, semi0),
                (srcc1, dstc1, xsr1, xdr1, exv1, semg1, semi1))
        lane = lax.iota(jnp.int32, L)
        base = wid * E_PER_W

        def start_idx(it, b):
            srcc, dstc, _, _, _, _, semi = bufs[b]
            off = base + it * C
            pltpu.async_copy(src_hbm.at[pl.ds(off, C)], srcc, semi)
            pltpu.async_copy(dst_hbm.at[pl.ds(off, C)], dstc, semi)

        def wait_idx(it, b):
            srcc, dstc, _, _, _, _, semi = bufs[b]
            off = base + it * C
            pltpu.make_async_copy(src_hbm.at[pl.ds(off, C)], srcc, semi).wait()
            pltpu.make_async_copy(dst_hbm.at[pl.ds(off, C)], dstc, semi).wait()

        def start_gather(b):
            srcc, dstc, xsr, xdr, _, semg, _ = bufs[b]
            pltpu.async_copy(xs_hbm.at[srcc], xsr, semg)
            pltpu.async_copy(xd_hbm.at[dstc], xdr, semg)

        def wait_gather(b):
            srcc, dstc, xsr, xdr, _, semg, _ = bufs[b]
            pltpu.make_async_copy(xs_hbm.at[srcc], xsr, semg).wait()
            pltpu.make_async_copy(xd_hbm.at[dstc], xdr, semg).wait()

        def process(b):
            _, dstc, xsr, xdr, exv, _, _ = bufs[b]

            hi_mask = jnp.full((L,), -65536, jnp.int32)

            def group_body(g, c2):
                e16 = jnp.zeros((L,), jnp.float32)
                for k in range(L):
                    i = g * L + k
                    acc16 = jnp.zeros((L,), jnp.float32)
                    for q in range(D // 32):
                        w = xdr[i, pl.ds(q * L, L)]
                        v0 = plsc.bitcast(w << 16, jnp.float32)
                        v1 = plsc.bitcast(w & hi_mask, jnp.float32)
                        for h, v in ((0, v0), (1, v1)):
                            dd = 32 * q + 16 * h
                            m = xsr[i, pl.ds(dd, L)] + v
                            m = jnp.where(m > 0.0, m, 0.2 * m)
                            acc16 = acc16 + m * att_v[pl.ds(dd, L)]
                    e16 = jnp.where(lane == k, jnp.sum(acc16), e16)
                ex16 = jnp.exp(e16)
                exv[pl.ds(g * L, L)] = ex16
                for k in range(L):
                    s = ex16[k]
                    i = g * L + k
                    for j in range(D // L):
                        xsr[i, pl.ds(j * L, L)] = xsr[i, pl.ds(j * L, L)] * s
                return c2

            lax.fori_loop(0, C // L, group_body, 0)

            pltpu.sync_copy(xsr, acc_sh.at[dstc], add=True)
            pltpu.sync_copy(exv, den_sh.at[dstc], add=True)

        start_idx(0, 0)
        wait_idx(0, 0)
        start_gather(0)
        start_idx(1, 1)

        zv = jnp.zeros((L,), jnp.float32)

        def zrow(i, carry):
            for j in range(D // L):
                xsr1[i, pl.ds(j * L, L)] = zv
            return carry

        lax.fori_loop(0, C, zrow, 0)

        def zd(i, carry):
            exv1[pl.ds(i * L, L)] = zv
            return carry

        lax.fori_loop(0, C // L, zd, 0)

        for kk in range(N_PER_TILE // C):
            pltpu.sync_copy(xsr1, acc_sh.at[pl.ds(sid * N_PER_TILE + kk * C, C)])
        pltpu.sync_copy(xsr1.at[pl.ds(0, N_PER_TILE % C)],
                        acc_sh.at[pl.ds(sid * N_PER_TILE
                                        + (N_PER_TILE // C) * C,
                                        N_PER_TILE % C)])
        for kk in range(ZDEN // C):
            pltpu.sync_copy(exv1, den_sh.at[pl.ds(sid * ZDEN + kk * C, C)])
        pltpu.sync_copy(att_hbm, att_v)
        plsc.subcore_barrier()

        def phase(it, b):
            wait_idx(it + 1, 1 - b)
            start_gather(1 - b)
            wait_gather(b)
            process(b)

            @pl.when(it + 2 < CHUNKS)
            def _():
                start_idx(it + 2, b)

        def pair_body(p, carry):
            it0 = 2 * p
            phase(it0, 0)
            phase(it0 + 1, 1)
            return carry

        lax.fori_loop(0, (CHUNKS - 1) // 2, pair_body, 0)
        wait_gather(0)
        process(0)

        plsc.subcore_barrier()

        @pl.when(sid < NS - 1)
        def _copy_out_main():
            pltpu.sync_copy(
                acc_sh.at[pl.ds(sid * 632, 632)],
                acc_out.at[cid, pl.ds(sid * 632, 632)])

        @pl.when(sid == NS - 1)
        def _copy_out_tail():
            pltpu.sync_copy(
                acc_sh.at[pl.ds((NS - 1) * 632, N - (NS - 1) * 632)],
                acc_out.at[cid, pl.ds((NS - 1) * 632, N - (NS - 1) * 632)])

        pltpu.sync_copy(
            den_sh.at[pl.ds(sid * ZDEN, ZDEN)],
            den_out.at[cid, pl.ds(sid * ZDEN, ZDEN)])

    return edge_kernel(xs, xd, src3, dst3, att)


def _finalize(acc, den, bias, gamma, beta):
    BLK = 1000

    def body(acc_ref, den_ref, b_ref, g_ref, bt_ref, o_ref):
        a = acc_ref[0] + acc_ref[1]
        dn = den_ref[0] + den_ref[1]
        out = a / (dn + 1e-16) + b_ref[...]
        mu = jnp.mean(out, axis=-1, keepdims=True)
        var = jnp.mean((out - mu) ** 2, axis=-1, keepdims=True)
        h = (out - mu) * lax.rsqrt(var + 1e-5)
        o_ref[...] = h * g_ref[...] + bt_ref[...]

    return pl.pallas_call(
        body,
        grid=(N // BLK,),
        in_specs=[
            pl.BlockSpec((2, BLK, D), lambda i: (0, i, 0)),
            pl.BlockSpec((2, BLK, 1), lambda i: (0, i, 0)),
            pl.BlockSpec((1, D), lambda i: (0, 0)),
            pl.BlockSpec((1, D), lambda i: (0, 0)),
            pl.BlockSpec((1, D), lambda i: (0, 0)),
        ],
        out_specs=pl.BlockSpec((BLK, D), lambda i: (i, 0)),
        out_shape=jax.ShapeDtypeStruct((N, D), jnp.float32),
    )(acc, den, bias, gamma, beta)


def kernel(x, edge_index, W_src, W_dst, att, bias, gamma, beta):
    xs, xd = _project(x, W_src, W_dst[:, jnp.asarray(_PERM)])
    xdi = lax.bitcast_convert_type(
        xd.astype(jnp.bfloat16).reshape(N, D // 2, 2), jnp.int32)
    acc, den = _sc_edge_pass(xs, xdi, edge_index[0], edge_index[1], att)
    den3 = den.reshape(NC, DEN_PAD, 1)
    return _finalize(acc, den3, bias.reshape(1, D),
                     gamma.reshape(1, D), beta.reshape(1, D))


# speedup vs baseline: 1.2165x; 1.2165x over previous
import functools

import jax
import jax.numpy as jnp
from jax import lax
from jax.experimental import pallas as pl
from jax.experimental.pallas import tpu as pltpu
from jax.experimental.pallas import tpu_sc as plsc

N = 10000
E = 320000
D = 128
L = 16
NC = 2
NS = 16
NW = NC * NS
E_PER_W = E // NW
C = 80
CHUNKS = E_PER_W // C
N_PER_TILE = N // NS
DEN_PAD = 10240
ZDEN = DEN_PAD // NS


def _project(x, W_src, W_dst):
    BLK = 1000

    def body(x_ref, ws_ref, wd_ref, xs_ref, xd_ref):
        xb = x_ref[...]
        xs_ref[...] = jnp.dot(xb, ws_ref[...], preferred_element_type=jnp.float32)
        xd_ref[...] = jnp.dot(xb, wd_ref[...], preferred_element_type=jnp.float32)

    return pl.pallas_call(
        body,
        grid=(N // BLK,),
        in_specs=[
            pl.BlockSpec((BLK, D), lambda i: (i, 0)),
            pl.BlockSpec((D, D), lambda i: (0, 0)),
            pl.BlockSpec((D, D), lambda i: (0, 0)),
        ],
        out_specs=[
            pl.BlockSpec((BLK, D), lambda i: (i, 0)),
            pl.BlockSpec((BLK, D), lambda i: (i, 0)),
        ],
        out_shape=[jax.ShapeDtypeStruct((N, D), jnp.float32)] * 2,
    )(x, W_src, W_dst)


def _sc_edge_pass(xs, xd, src3, dst3, att):
    mesh = plsc.VectorSubcoreMesh(core_axis_name="c", subcore_axis_name="s")

    @functools.partial(
        pl.kernel,
        out_type=[
            jax.ShapeDtypeStruct((NC, N, D), jnp.float32),
            jax.ShapeDtypeStruct((NC, DEN_PAD), jnp.float32),
        ],
        mesh=mesh,
        compiler_params=pltpu.CompilerParams(needs_layout_passes=False),
        scratch_types=[
            pltpu.VMEM((C,), jnp.int32),
            pltpu.VMEM((C,), jnp.int32),
            pltpu.VMEM((C,), jnp.int32),
            pltpu.VMEM((C,), jnp.int32),
            pltpu.VMEM((C, D), jnp.float32),
            pltpu.VMEM((C, D), jnp.float32),
            pltpu.VMEM((C, D), jnp.float32),
            pltpu.VMEM((C, D), jnp.float32),
            pltpu.VMEM((C,), jnp.float32),
            pltpu.VMEM((C,), jnp.float32),
            pltpu.VMEM((D,), jnp.float32),
            pltpu.VMEM((D,), jnp.float32),
            pltpu.VMEM_SHARED((N, D), jnp.float32),
            pltpu.VMEM_SHARED((DEN_PAD,), jnp.float32),
            pltpu.SemaphoreType.DMA,
            pltpu.SemaphoreType.DMA,
            pltpu.SemaphoreType.DMA,
            pltpu.SemaphoreType.DMA,
        ],
    )
    def edge_kernel(xs_hbm, xd_hbm, src_hbm, dst_hbm, att6_hbm, att4_hbm,
                    acc_out, den_out,
                    srcc0, srcc1, dstc0, dstc1, xsr0, xsr1, xdr0, xdr1,
                    exv0, exv1, att6_v, att4_v, acc_sh, den_sh,
                    semg0, semg1, semi0, semi1):
        cid = lax.axis_index("c")
        sid = lax.axis_index("s")
        wid = sid * NC + cid

        bufs = ((srcc0, dstc0, xsr0, xdr0, exv0, semg0, semi0),
                (srcc1, dstc1, xsr1, xdr1, exv1, semg1, semi1))
        lane = lax.iota(jnp.int32, L)
        base = wid * E_PER_W

        def start_idx(it, b):
            srcc, dstc, _, _, _, _, semi = bufs[b]
            off = base + it * C
            pltpu.async_copy(src_hbm.at[pl.ds(off, C)], srcc, semi)
            pltpu.async_copy(dst_hbm.at[pl.ds(off, C)], dstc, semi)

        def wait_idx(it, b):
            srcc, dstc, _, _, _, _, semi = bufs[b]
            off = base + it * C
            pltpu.make_async_copy(src_hbm.at[pl.ds(off, C)], srcc, semi).wait()
            pltpu.make_async_copy(dst_hbm.at[pl.ds(off, C)], dstc, semi).wait()

        def start_gather(b):
            srcc, dstc, xsr, xdr, _, semg, _ = bufs[b]
            pltpu.async_copy(xs_hbm.at[srcc], xsr, semg)
            pltpu.async_copy(xd_hbm.at[dstc], xdr, semg)

        def wait_gather(b):
            srcc, dstc, xsr, xdr, _, semg, _ = bufs[b]
            pltpu.make_async_copy(xs_hbm.at[srcc], xsr, semg).wait()
            pltpu.make_async_copy(xd_hbm.at[dstc], xdr, semg).wait()

        def process(b):
            _, dstc, xsr, xdr, exv, _, _ = bufs[b]

            att6 = [att6_v[pl.ds(j * L, L)] for j in range(D // L)]
            att4 = [att4_v[pl.ds(j * L, L)] for j in range(D // L)]

            def group_body(g, c2):
                e16 = jnp.zeros((L,), jnp.float32)
                for k in range(L):
                    i = g * L + k
                    acc16 = jnp.zeros((L,), jnp.float32)
                    for j in range(D // L):
                        a = xsr[i, pl.ds(j * L, L)]
                        b2 = xdr[i, pl.ds(j * L, L)]
                        m = a + b2
                        acc16 = acc16 + m * att6[j] + jnp.abs(m) * att4[j]
                    e16 = jnp.where(lane == k, jnp.sum(acc16), e16)
                ex16 = jnp.exp(e16)
                exv[pl.ds(g * L, L)] = ex16
                for k in range(L):
                    s = ex16[k]
                    i = g * L + k
                    for j in range(D // L):
                        xsr[i, pl.ds(j * L, L)] = xsr[i, pl.ds(j * L, L)] * s
                return c2

            lax.fori_loop(0, C // L, group_body, 0)

            pltpu.sync_copy(xsr, acc_sh.at[dstc], add=True)
            pltpu.sync_copy(exv, den_sh.at[dstc], add=True)

        start_idx(0, 0)
        wait_idx(0, 0)
        start_gather(0)
        start_idx(1, 1)

        zv = jnp.zeros((L,), jnp.float32)

        def zrow(i, carry):
            for j in range(D // L):
                xsr1[i, pl.ds(j * L, L)] = zv
            return carry

        lax.fori_loop(0, C, zrow, 0)

        def zd(i, carry):
            exv1[pl.ds(i * L, L)] = zv
            return carry

        lax.fori_loop(0, C // L, zd, 0)

        for kk in range(N_PER_TILE // C):
            pltpu.sync_copy(xsr1, acc_sh.at[pl.ds(sid * N_PER_TILE + kk * C, C)])
        pltpu.sync_copy(xsr1.at[pl.ds(0, N_PER_TILE % C)],
                        acc_sh.at[pl.ds(sid * N_PER_TILE
                                        + (N_PER_TILE // C) * C,
                                        N_PER_TILE % C)])
        for kk in range(ZDEN // C):
            pltpu.sync_copy(exv1, den_sh.at[pl.ds(sid * ZDEN + kk * C, C)])
        pltpu.sync_copy(att6_hbm, att6_v)
        pltpu.sync_copy(att4_hbm, att4_v)
        plsc.subcore_barrier()

        def phase(it, b):
            wait_idx(it + 1, 1 - b)
            start_gather(1 - b)
            wait_gather(b)
            process(b)

            @pl.when(it + 2 < CHUNKS)
            def _():
                start_idx(it + 2, b)

        def pair_body(p, carry):
            it0 = 2 * p
            phase(it0, 0)
            phase(it0 + 1, 1)
            return carry

        lax.fori_loop(0, (CHUNKS - 1) // 2, pair_body, 0)
        wait_gather(0)
        process(0)

        plsc.subcore_barrier()

        @pl.when(sid < NS - 1)
        def _copy_out_main():
            pltpu.sync_copy(
                acc_sh.at[pl.ds(sid * 632, 632)],
                acc_out.at[cid, pl.ds(sid * 632, 632)])

        @pl.when(sid == NS - 1)
        def _copy_out_tail():
            pltpu.sync_copy(
                acc_sh.at[pl.ds((NS - 1) * 632, N - (NS - 1) * 632)],
                acc_out.at[cid, pl.ds((NS - 1) * 632, N - (NS - 1) * 632)])

        pltpu.sync_copy(
            den_sh.at[pl.ds(sid * ZDEN, ZDEN)],
            den_out.at[cid, pl.ds(sid * ZDEN, ZDEN)])

    return edge_kernel(xs, xd, src3, dst3, 0.6 * att, 0.4 * att)


def _finalize(acc, den, bias, gamma, beta):
    BLK = 1000

    def body(acc_ref, den_ref, b_ref, g_ref, bt_ref, o_ref):
        a = acc_ref[0] + acc_ref[1]
        dn = den_ref[0] + den_ref[1]
        out = a / (dn + 1e-16) + b_ref[...]
        mu = jnp.mean(out, axis=-1, keepdims=True)
        var = jnp.mean((out - mu) ** 2, axis=-1, keepdims=True)
        h = (out - mu) * lax.rsqrt(var + 1e-5)
        o_ref[...] = h * g_ref[...] + bt_ref[...]

    return pl.pallas_call(
        body,
        grid=(N // BLK,),
        in_specs=[
            pl.BlockSpec((2, BLK, D), lambda i: (0, i, 0)),
            pl.BlockSpec((2, BLK, 1), lambda i: (0, i, 0)),
            pl.BlockSpec((1, D), lambda i: (0, 0)),
            pl.BlockSpec((1, D), lambda i: (0, 0)),
            pl.BlockSpec((1, D), lambda i: (0, 0)),
        ],
        out_specs=pl.BlockSpec((BLK, D), lambda i: (i, 0)),
        out_shape=jax.ShapeDtypeStruct((N, D), jnp.float32),
    )(acc, den, bias, gamma, beta)


def kernel(x, edge_index, W_src, W_dst, att, bias, gamma, beta):
    xs, xd = _project(x, W_src, W_dst)
    acc, den = _sc_edge_pass(xs, xd, edge_index[0], edge_index[1], att)
    den3 = den.reshape(NC, DEN_PAD, 1)
    return _finalize(acc, den3, bias.reshape(1, D),
                     gamma.reshape(1, D), beta.reshape(1, D))


# speedup vs baseline: 1.2440x; 1.0226x over previous
import functools

import jax
import jax.numpy as jnp
from jax import lax
from jax.experimental import pallas as pl
from jax.experimental.pallas import tpu as pltpu
from jax.experimental.pallas import tpu_sc as plsc

N = 10000
E = 320000
D = 128
L = 16
NC = 2
NS = 16
NW = NC * NS
E_PER_W = E // NW
C = 80
CHUNKS = E_PER_W // C
N_PER_TILE = N // NS
DEN_PAD = 10240
ZDEN = DEN_PAD // NS


def _project(x, W_src, W_dst):
    BLK = 1000

    def body(x_ref, ws_ref, wd_ref, xs_ref, xd_ref):
        xb = x_ref[...]
        xs_ref[...] = jnp.dot(xb, ws_ref[...], preferred_element_type=jnp.float32)
        xd_ref[...] = jnp.dot(xb, wd_ref[...], preferred_element_type=jnp.float32)

    return pl.pallas_call(
        body,
        grid=(N // BLK,),
        in_specs=[
            pl.BlockSpec((BLK, D), lambda i: (i, 0)),
            pl.BlockSpec((D, D), lambda i: (0, 0)),
            pl.BlockSpec((D, D), lambda i: (0, 0)),
        ],
        out_specs=[
            pl.BlockSpec((BLK, D), lambda i: (i, 0)),
            pl.BlockSpec((BLK, D), lambda i: (i, 0)),
        ],
        out_shape=[jax.ShapeDtypeStruct((N, D), jnp.float32)] * 2,
    )(x, W_src, W_dst)


def _sc_edge_pass(xs, xd, src3, dst3, att):
    mesh = plsc.VectorSubcoreMesh(core_axis_name="c", subcore_axis_name="s")

    @functools.partial(
        pl.kernel,
        out_type=[
            jax.ShapeDtypeStruct((NC, N, D), jnp.float32),
            jax.ShapeDtypeStruct((NC, DEN_PAD), jnp.float32),
        ],
        mesh=mesh,
        compiler_params=pltpu.CompilerParams(needs_layout_passes=False),
        scratch_types=[
            pltpu.VMEM((C,), jnp.int32),
            pltpu.VMEM((C,), jnp.int32),
            pltpu.VMEM((C,), jnp.int32),
            pltpu.VMEM((C,), jnp.int32),
            pltpu.VMEM((C, D), jnp.float32),
            pltpu.VMEM((C, D), jnp.float32),
            pltpu.VMEM((C, D), jnp.float32),
            pltpu.VMEM((C, D), jnp.float32),
            pltpu.VMEM((C,), jnp.float32),
            pltpu.VMEM((C,), jnp.float32),
            pltpu.VMEM((D,), jnp.float32),
            pltpu.VMEM((D,), jnp.float32),
            pltpu.VMEM_SHARED((N, D), jnp.float32),
            pltpu.VMEM_SHARED((DEN_PAD,), jnp.float32),
            pltpu.SemaphoreType.DMA,
            pltpu.SemaphoreType.DMA,
            pltpu.SemaphoreType.DMA,
            pltpu.SemaphoreType.DMA,
        ],
    )
    def edge_kernel(xs_hbm, xd_hbm, src_hbm, dst_hbm, att6_hbm, att4_hbm,
                    acc_out, den_out,
                    srcc0, srcc1, dstc0, dstc1, xsr0, xsr1, xdr0, xdr1,
                    exv0, exv1, att6_v, att4_v, acc_sh, den_sh,
                    semg0, semg1, semi0, semi1):
        cid = lax.axis_index("c")
        sid = lax.axis_index("s")
        wid = sid * NC + cid

        bufs = ((srcc0, dstc0, xsr0, xdr0, exv0, semg0, semi0),
                (srcc1, dstc1, xsr1, xdr1, exv1, semg1, semi1))
        lane = lax.iota(jnp.int32, L)
        base = wid * E_PER_W

        def start_idx(it, b):
            srcc, dstc, _, _, _, _, semi = bufs[b]
            off = base + it * C
            pltpu.async_copy(src_hbm.at[pl.ds(off, C)], srcc, semi)
            pltpu.async_copy(dst_hbm.at[pl.ds(off, C)], dstc, semi)

        def wait_idx(it, b):
            srcc, dstc, _, _, _, _, semi = bufs[b]
            off = base + it * C
            pltpu.make_async_copy(src_hbm.at[pl.ds(off, C)], srcc, semi).wait()
            pltpu.make_async_copy(dst_hbm.at[pl.ds(off, C)], dstc, semi).wait()

        def start_gather(b):
            srcc, dstc, xsr, xdr, _, semg, _ = bufs[b]
            pltpu.async_copy(xs_hbm.at[srcc], xsr, semg)
            pltpu.async_copy(xd_hbm.at[dstc], xdr, semg)

        def wait_gather(b):
            srcc, dstc, xsr, xdr, _, semg, _ = bufs[b]
            pltpu.make_async_copy(xs_hbm.at[srcc], xsr, semg).wait()
            pltpu.make_async_copy(xd_hbm.at[dstc], xdr, semg).wait()

        def process(b):
            _, dstc, xsr, xdr, exv, semg, _ = bufs[b]

            att6 = [att6_v[pl.ds(j * L, L)] for j in range(D // L)]
            att4 = [att4_v[pl.ds(j * L, L)] for j in range(D // L)]

            def group_body(g, c2):
                e16 = jnp.zeros((L,), jnp.float32)
                for k in range(L):
                    i = g * L + k
                    acc16 = jnp.zeros((L,), jnp.float32)
                    for j in range(D // L):
                        a = xsr[i, pl.ds(j * L, L)]
                        b2 = xdr[i, pl.ds(j * L, L)]
                        m = a + b2
                        acc16 = acc16 + m * att6[j] + jnp.abs(m) * att4[j]
                    e16 = jnp.where(lane == k, jnp.sum(acc16), e16)
                ex16 = jnp.exp(e16)
                exv[pl.ds(g * L, L)] = ex16
                for k in range(L):
                    s = ex16[k]
                    i = g * L + k
                    for j in range(D // L):
                        xsr[i, pl.ds(j * L, L)] = xsr[i, pl.ds(j * L, L)] * s
                return c2

            lax.fori_loop(0, C // L, group_body, 0)

            cp1 = pltpu.async_copy(xsr, acc_sh.at[dstc], semg, add=True)
            cp2 = pltpu.async_copy(exv, den_sh.at[dstc], semg, add=True)
            cp1.wait()
            cp2.wait()

        start_idx(0, 0)
        wait_idx(0, 0)
        start_gather(0)
        start_idx(1, 1)

        zv = jnp.zeros((L,), jnp.float32)

        def zrow(i, carry):
            for j in range(D // L):
                xsr1[i, pl.ds(j * L, L)] = zv
            return carry

        lax.fori_loop(0, C, zrow, 0)

        def zd(i, carry):
            exv1[pl.ds(i * L, L)] = zv
            return carry

        lax.fori_loop(0, C // L, zd, 0)

        for kk in range(N_PER_TILE // C):
            pltpu.sync_copy(xsr1, acc_sh.at[pl.ds(sid * N_PER_TILE + kk * C, C)])
        pltpu.sync_copy(xsr1.at[pl.ds(0, N_PER_TILE % C)],
                        acc_sh.at[pl.ds(sid * N_PER_TILE
                                        + (N_PER_TILE // C) * C,
                                        N_PER_TILE % C)])
        for kk in range(ZDEN // C):
            pltpu.sync_copy(exv1, den_sh.at[pl.ds(sid * ZDEN + kk * C, C)])
        pltpu.sync_copy(att6_hbm, att6_v)
        pltpu.sync_copy(att4_hbm, att4_v)
        plsc.subcore_barrier()

        def phase(it, b):
            wait_idx(it + 1, 1 - b)
            start_gather(1 - b)
            wait_gather(b)
            process(b)

            @pl.when(it + 2 < CHUNKS)
            def _():
                start_idx(it + 2, b)

        def pair_body(p, carry):
            it0 = 2 * p
            phase(it0, 0)
            phase(it0 + 1, 1)
            return carry

        lax.fori_loop(0, (CHUNKS - 1) // 2, pair_body, 0)
        wait_gather(0)
        process(0)

        plsc.subcore_barrier()

        @pl.when(sid < NS - 1)
        def _copy_out_main():
            pltpu.sync_copy(
                acc_sh.at[pl.ds(sid * 632, 632)],
                acc_out.at[cid, pl.ds(sid * 632, 632)])

        @pl.when(sid == NS - 1)
        def _copy_out_tail():
            pltpu.sync_copy(
                acc_sh.at[pl.ds((NS - 1) * 632, N - (NS - 1) * 632)],
                acc_out.at[cid, pl.ds((NS - 1) * 632, N - (NS - 1) * 632)])

        pltpu.sync_copy(
            den_sh.at[pl.ds(sid * ZDEN, ZDEN)],
            den_out.at[cid, pl.ds(sid * ZDEN, ZDEN)])

    return edge_kernel(xs, xd, src3, dst3, 0.6 * att, 0.4 * att)


def _finalize(acc, den, bias, gamma, beta):
    BLK = 1000

    def body(acc_ref, den_ref, b_ref, g_ref, bt_ref, o_ref):
        a = acc_ref[0] + acc_ref[1]
        dn = den_ref[0] + den_ref[1]
        out = a / (dn + 1e-16) + b_ref[...]
        mu = jnp.mean(out, axis=-1, keepdims=True)
        var = jnp.mean((out - mu) ** 2, axis=-1, keepdims=True)
        h = (out - mu) * lax.rsqrt(var + 1e-5)
        o_ref[...] = h * g_ref[...] + bt_ref[...]

    return pl.pallas_call(
        body,
        grid=(N // BLK,),
        in_specs=[
            pl.BlockSpec((2, BLK, D), lambda i: (0, i, 0)),
            pl.BlockSpec((2, BLK, 1), lambda i: (0, i, 0)),
            pl.BlockSpec((1, D), lambda i: (0, 0)),
            pl.BlockSpec((1, D), lambda i: (0, 0)),
            pl.BlockSpec((1, D), lambda i: (0, 0)),
        ],
        out_specs=pl.BlockSpec((BLK, D), lambda i: (i, 0)),
        out_shape=jax.ShapeDtypeStruct((N, D), jnp.float32),
    )(acc, den, bias, gamma, beta)


def kernel(x, edge_index, W_src, W_dst, att, bias, gamma, beta):
    xs, xd = _project(x, W_src, W_dst)
    acc, den = _sc_edge_pass(xs, xd, edge_index[0], edge_index[1], att)
    den3 = den.reshape(NC, DEN_PAD, 1)
    return _finalize(acc, den3, bias.reshape(1, D),
                     gamma.reshape(1, D), beta.reshape(1, D))
